# baseline (speedup 1.0000x reference)
import functools

import jax
import jax.numpy as jnp
from jax import lax
from jax.experimental import pallas as pl
from jax.experimental.pallas import tpu as pltpu
from jax.experimental.pallas import tpu_sc as plsc

_B = 16384
_H = 128
_ITEM_NUM = 100000

_info = plsc.get_sparse_core_info()
_NC = _info.num_cores
_NS = _info.num_subcores
_NW = _NC * _NS
_BPW = _B // _NW
_TCH = 128

_SG_ROWS = (2 * _ITEM_NUM + 127) // 128

_mesh = plsc.VectorSubcoreMesh(core_axis_name="c", subcore_axis_name="s")


@functools.partial(
    pl.kernel,
    mesh=_mesh,
    compiler_params=pltpu.CompilerParams(needs_layout_passes=False),
    out_type=[
        jax.ShapeDtypeStruct((_B, _H), jnp.float32),
        jax.ShapeDtypeStruct((_B,), jnp.float32),
        jax.ShapeDtypeStruct((_B,), jnp.float32),
    ],
    scratch_types=[
        pltpu.VMEM((_BPW,), jnp.int32),
        pltpu.VMEM((_BPW,), jnp.int32),
        pltpu.VMEM((_BPW,), jnp.int32),
        pltpu.VMEM((2, _TCH, _H), jnp.float32),
        pltpu.VMEM((_BPW, _H), jnp.float32),
        pltpu.VMEM((_BPW,), jnp.float32),
        pltpu.VMEM((_BPW,), jnp.float32),
        pltpu.SemaphoreType.DMA,
        pltpu.SemaphoreType.DMA,
    ],
)
def _sc_gather(user_hbm, item_hbm, theta_hbm, sg_hbm,
               theta_out, slip_out, guess_out,
               uidx_v, iidx_v, ridx_v, rows_v, sg_rows_v, slip_v, guess_v,
               sem_t, sem_sg):
    wid = lax.axis_index("s") * _NC + lax.axis_index("c")
    base = wid * _BPW
    pltpu.sync_copy(user_hbm.at[pl.ds(base, _BPW)], uidx_v)
    pltpu.sync_copy(item_hbm.at[pl.ds(base, _BPW)], iidx_v)

    for i in range(_BPW // 16):
        v = iidx_v[pl.ds(i * 16, 16)]
        ridx_v[pl.ds(i * 16, 16)] = lax.shift_right_logical(v, 6)

    c_sg = pltpu.async_copy(sg_hbm.at[ridx_v], sg_rows_v, sem_sg)
    c_t0 = pltpu.async_copy(
        theta_hbm.at[uidx_v.at[pl.ds(0, _TCH)]], rows_v.at[0], sem_t)

    c_sg.wait()
    lane = lax.iota(jnp.int32, 16)
    for i in range(_BPW // 16):
        v = iidx_v[pl.ds(i * 16, 16)]
        col = lax.shift_left(lax.bitwise_and(v, 63), 1)
        row = jnp.full((16,), i * 16, jnp.int32) + lane
        slip_v[pl.ds(i * 16, 16)] = plsc.load_gather(sg_rows_v, [row, col])
        guess_v[pl.ds(i * 16, 16)] = plsc.load_gather(
            sg_rows_v, [row, col + 1])
    pltpu.sync_copy(slip_v, slip_out.at[pl.ds(base, _BPW)])
    pltpu.sync_copy(guess_v, guess_out.at[pl.ds(base, _BPW)])

    n_chunks = _BPW // _TCH
    for c in range(n_chunks):
        cur = c % 2
        if c + 1 < n_chunks:
            nxt_copy = pltpu.async_copy(
                theta_hbm.at[uidx_v.at[pl.ds((c + 1) * _TCH, _TCH)]],
                rows_v.at[(c + 1) % 2], sem_t)
        if c == 0:
            c_t0.wait()
        else:
            prev_wait.wait()
        pltpu.sync_copy(rows_v.at[cur],
                        theta_out.at[pl.ds(base + c * _TCH, _TCH)])
        if c + 1 < n_chunks:
            prev_wait = nxt_copy


_BM = 2048


def _tc_body(theta_ref, know_ref, slip_ref, guess_ref, diff_ref, w_ref,
             b_ref, out_ref):
    theta = theta_ref[...]
    know = know_ref[...]
    n = jnp.sum(know * (jax.nn.sigmoid(theta) - 0.5), axis=1, keepdims=True)
    p = jax.nn.sigmoid(n * (1.0 / 50.0))
    slip = jax.nn.sigmoid(slip_ref[...]) * 0.4
    guess = jax.nn.sigmoid(guess_ref[...]) * 0.4
    scores = (1.0 - slip) * p + guess * (1.0 - p)
    out = scores * diff_ref[...] * w_ref[0, 0] + b_ref[0, 0]
    out_ref[...] = jax.nn.sigmoid(out)


def _tc_dense(theta_g, knowledge, slip_g, guess_g, diff2, out_w, out_b2):
    grid = (_B // _BM,)
    return pl.pallas_call(
        _tc_body,
        grid=grid,
        in_specs=[
            pl.BlockSpec((_BM, _H), lambda i: (i, 0)),
            pl.BlockSpec((_BM, _H), lambda i: (i, 0)),
            pl.BlockSpec((_BM, 1), lambda i: (i, 0)),
            pl.BlockSpec((_BM, 1), lambda i: (i, 0)),
            pl.BlockSpec((_BM, 1), lambda i: (i, 0)),
            pl.BlockSpec((1, 1), lambda i: (0, 0)),
            pl.BlockSpec((1, 1), lambda i: (0, 0)),
        ],
        out_specs=pl.BlockSpec((_BM, 1), lambda i: (i, 0)),
        out_shape=jax.ShapeDtypeStruct((_B, 1), jnp.float32),
    )(theta_g, knowledge, slip_g, guess_g, diff2, out_w, out_b2)


def kernel(user, item, knowledge, diff, theta_w, slip_w, guess_w, out_w,
           out_b):
    sg = jnp.stack([slip_w[:, 0], guess_w[:, 0]], axis=1).reshape(-1)
    sg = jnp.pad(sg, (0, _SG_ROWS * 128 - sg.shape[0])).reshape(_SG_ROWS, 128)
    theta_g, slip_g, guess_g = _sc_gather(user, item, theta_w, sg)
    out = _tc_dense(theta_g, knowledge, slip_g.reshape(_B, 1),
                    guess_g.reshape(_B, 1), diff.reshape(_B, 1), out_w,
                    out_b.reshape(1, 1))
    return out.reshape(_B)


# speedup vs baseline: 2.5113x; 2.5113x over previous
import functools

import jax
import jax.numpy as jnp
from jax import lax
from jax.experimental import pallas as pl
from jax.experimental.pallas import tpu as pltpu
from jax.experimental.pallas import tpu_sc as plsc

_B = 16384
_H = 128
_ITEM_NUM = 100000

_info = plsc.get_sparse_core_info()
_NC = _info.num_cores
_NS = _info.num_subcores
_NW = _NC * _NS
_BPW = _B // _NW
_TCH = 128

_mesh = plsc.VectorSubcoreMesh(core_axis_name="c", subcore_axis_name="s")


@functools.partial(
    pl.kernel,
    mesh=_mesh,
    compiler_params=pltpu.CompilerParams(
        needs_layout_passes=False, use_tc_tiling_on_sc=False),
    out_type=[
        jax.ShapeDtypeStruct((_B, _H), jnp.float32),
        jax.ShapeDtypeStruct((_B,), jnp.float32),
        jax.ShapeDtypeStruct((_B,), jnp.float32),
    ],
    scratch_types=[
        pltpu.VMEM((_BPW,), jnp.int32),
        pltpu.VMEM((_BPW,), jnp.int32),
        pltpu.VMEM((2, _TCH, _H), jnp.float32),
        pltpu.VMEM((_BPW,), jnp.float32),
        pltpu.VMEM((_BPW,), jnp.float32),
        pltpu.SemaphoreType.DMA,
        pltpu.SemaphoreType.DMA,
    ],
)
def _sc_gather(user_hbm, item_hbm, theta_hbm, slip_hbm, guess_hbm,
               theta_out, slip_out, guess_out,
               uidx_v, iidx_v, rows_v, slip_v, guess_v,
               sem_t, sem_sg):
    wid = lax.axis_index("s") * _NC + lax.axis_index("c")
    base = wid * _BPW
    pltpu.sync_copy(user_hbm.at[pl.ds(base, _BPW)], uidx_v)
    pltpu.sync_copy(item_hbm.at[pl.ds(base, _BPW)], iidx_v)

    c_s = pltpu.async_copy(slip_hbm.at[iidx_v], slip_v, sem_sg)
    c_g = pltpu.async_copy(guess_hbm.at[iidx_v], guess_v, sem_sg)
    c_t0 = pltpu.async_copy(
        theta_hbm.at[uidx_v.at[pl.ds(0, _TCH)]], rows_v.at[0], sem_t)
    c_s.wait()
    c_g.wait()
    pltpu.sync_copy(slip_v, slip_out.at[pl.ds(base, _BPW)])
    pltpu.sync_copy(guess_v, guess_out.at[pl.ds(base, _BPW)])

    n_chunks = _BPW // _TCH
    for c in range(n_chunks):
        cur = c % 2
        if c + 1 < n_chunks:
            nxt_copy = pltpu.async_copy(
                theta_hbm.at[uidx_v.at[pl.ds((c + 1) * _TCH, _TCH)]],
                rows_v.at[(c + 1) % 2], sem_t)
        if c == 0:
            c_t0.wait()
        else:
            prev_wait.wait()
        pltpu.sync_copy(rows_v.at[cur],
                        theta_out.at[pl.ds(base + c * _TCH, _TCH)])
        if c + 1 < n_chunks:
            prev_wait = nxt_copy


_BM = 2048


def _tc_body(theta_ref, know_ref, slip_ref, guess_ref, diff_ref, w_ref,
             b_ref, out_ref):
    theta = theta_ref[...]
    know = know_ref[...]
    n = jnp.sum(know * (jax.nn.sigmoid(theta) - 0.5), axis=1)
    p = jax.nn.sigmoid(n * (1.0 / 50.0))
    slip = jax.nn.sigmoid(slip_ref[...]) * 0.4
    guess = jax.nn.sigmoid(guess_ref[...]) * 0.4
    scores = (1.0 - slip) * p + guess * (1.0 - p)
    out = scores * diff_ref[...] * w_ref[0] + b_ref[0]
    out_ref[...] = jax.nn.sigmoid(out)


def _tc_dense(theta_g, knowledge, slip_g, guess_g, diff, out_w1, out_b):
    grid = (_B // _BM,)
    return pl.pallas_call(
        _tc_body,
        grid=grid,
        in_specs=[
            pl.BlockSpec((_BM, _H), lambda i: (i, 0)),
            pl.BlockSpec((_BM, _H), lambda i: (i, 0)),
            pl.BlockSpec((_BM,), lambda i: (i,)),
            pl.BlockSpec((_BM,), lambda i: (i,)),
            pl.BlockSpec((_BM,), lambda i: (i,)),
            pl.BlockSpec((1,), lambda i: (0,)),
            pl.BlockSpec((1,), lambda i: (0,)),
        ],
        out_specs=pl.BlockSpec((_BM,), lambda i: (i,)),
        out_shape=jax.ShapeDtypeStruct((_B,), jnp.float32),
    )(theta_g, knowledge, slip_g, guess_g, diff, out_w1, out_b)


def kernel(user, item, knowledge, diff, theta_w, slip_w, guess_w, out_w,
           out_b):
    theta_g, slip_g, guess_g = _sc_gather(user, item, theta_w,
                                          slip_w.reshape(_ITEM_NUM),
                                          guess_w.reshape(_ITEM_NUM))
    return _tc_dense(theta_g, knowledge, slip_g, guess_g, diff,
                     out_w.reshape(1), out_b)
